# baseline (speedup 1.0000x reference)
import functools

import jax
import jax.numpy as jnp
from jax import lax
from jax.experimental import pallas as pl
from jax.experimental.pallas import tpu as pltpu
from jax.experimental.pallas import tpu_sc as plsc

_NUM_CORES = 2
_NUM_SUBCORES = 16
_NUM_WORKERS = _NUM_CORES * _NUM_SUBCORES
_NBUF = 4
_CHUNK_ROWS = 16


@functools.lru_cache(maxsize=None)
def _make_copy_kernel(seq_len: int, dim: int):
    rows_per_w = seq_len // _NUM_WORKERS
    chunk = min(rows_per_w, _CHUNK_ROWS)
    nchunk = rows_per_w // chunk
    mesh = plsc.VectorSubcoreMesh(core_axis_name="c", subcore_axis_name="s")

    @functools.partial(
        pl.kernel,
        mesh=mesh,
        out_type=jax.ShapeDtypeStruct((seq_len, dim), jnp.float32),
        scratch_types=(
            [pltpu.VMEM((chunk, dim), jnp.float32) for _ in range(_NBUF)]
            + [pltpu.SemaphoreType.DMA for _ in range(2 * _NBUF)]
        ),
    )
    def k(emb_hbm, out_hbm, *scratch):
        bufs = scratch[:_NBUF]
        rsems = scratch[_NBUF:2 * _NBUF]
        wsems = scratch[2 * _NBUF:]
        wid = lax.axis_index("s") * _NUM_CORES + lax.axis_index("c")
        base = wid * rows_per_w

        def read(c):
            b = c % _NBUF
            return pltpu.async_copy(
                emb_hbm.at[pl.ds(base + c * chunk, chunk)], bufs[b], rsems[b])

        def write(c):
            b = c % _NBUF
            return pltpu.async_copy(
                bufs[b], out_hbm.at[pl.ds(base + c * chunk, chunk)], wsems[b])

        reads = {}
        writes = {}
        for c in range(min(_NBUF, nchunk)):
            reads[c] = read(c)
        for c in range(nchunk):
            reads.pop(c).wait()
            writes[c] = write(c)
            nxt = c + _NBUF
            if nxt < nchunk:
                writes.pop(nxt - _NBUF).wait()
                reads[nxt] = read(nxt)
        for w in writes.values():
            w.wait()

    return k


def kernel(x, emb_weight):
    seq_len = x.shape[1]
    dim = emb_weight.shape[1]
    return _make_copy_kernel(seq_len, dim)(emb_weight)
